# baseline (speedup 1.0000x reference)
import functools

import jax
import jax.numpy as jnp
from jax import lax
from jax.experimental import pallas as pl
from jax.experimental.pallas import tpu as pltpu, tpu_sc as plsc

B = 16384
DIM = 64
CH = 128
NBUF = 3


@functools.lru_cache(maxsize=None)
def _build(num_cores, num_subcores):
    NW = num_cores * num_subcores
    b_per_w = B // NW
    NCH = b_per_w // CH
    G = CH // 16

    mesh = plsc.VectorSubcoreMesh(core_axis_name="c", subcore_axis_name="s")
    out_sds = jax.ShapeDtypeStruct((B, DIM), jnp.float32)

    @functools.partial(
        pl.kernel,
        mesh=mesh,
        out_type=(out_sds, out_sds, out_sds),
        scratch_types=[
            pltpu.VMEM((b_per_w,), jnp.int32),
            pltpu.VMEM((b_per_w,), jnp.int32),
            pltpu.VMEM((b_per_w,), jnp.int32),
            pltpu.VMEM((b_per_w,), jnp.int32),
            pltpu.VMEM((b_per_w,), jnp.int32),
            pltpu.VMEM((b_per_w,), jnp.int32),
            [pltpu.VMEM((CH, 2 * DIM), jnp.float32) for _ in range(NBUF)],
            [pltpu.VMEM((CH, DIM), jnp.float32) for _ in range(NBUF)],
            [pltpu.SemaphoreType.DMA for _ in range(NBUF)],
            [pltpu.SemaphoreType.DMA for _ in range(NBUF)],
        ],
        compiler_params=pltpu.CompilerParams(needs_layout_passes=False),
    )
    def sc_gather3(u_p, u_h, p_p, p_h, n_p, n_h, utab, itab,
                   out_u, out_p, out_n,
                   upv, uhv, ppv, phv, npv, nhv, pairs, rows, gsems, wsems):
        wid = lax.axis_index("s") * num_cores + lax.axis_index("c")
        base = wid * b_per_w

        pltpu.sync_copy(u_p.at[wid], upv)
        pltpu.sync_copy(u_h.at[wid], uhv)
        pltpu.sync_copy(p_p.at[wid], ppv)
        pltpu.sync_copy(p_h.at[wid], phv)
        pltpu.sync_copy(n_p.at[wid], npv)
        pltpu.sync_copy(n_h.at[wid], nhv)

        sched = []
        for tab, pv, hv, out in ((utab, upv, uhv, out_u),
                                 (itab, ppv, phv, out_p),
                                 (itab, npv, nhv, out_n)):
            for c in range(NCH):
                sched.append((tab, pv, hv, out, c * CH))
        total = len(sched)

        def fire(slot):
            tab, pv, _, _, ofs = sched[slot]
            pltpu.async_copy(tab.at[pv.at[pl.ds(ofs, CH)]],
                             pairs[slot % NBUF], gsems[slot % NBUF])

        def drain_gather(slot):
            tab = sched[slot][0]
            pltpu.make_async_copy(tab.at[sched[slot][1].at[pl.ds(0, CH)]],
                                  pairs[slot % NBUF], gsems[slot % NBUF]).wait()

        def extract(slot):
            _, _, hv, _, ofs = sched[slot]
            pbuf = pairs[slot % NBUF]
            rbuf = rows[slot % NBUF]

            def group(g, carry):
                jvec = lax.iota(jnp.int32, 16)
                hvec = hv[pl.ds(ofs + g * 16, 16)]
                jrow = jvec + g * 16
                for col in range(DIM):
                    x = plsc.load_gather(pbuf, [jrow, hvec + col])
                    plsc.store_scatter(rbuf, [jrow, jnp.full((16,), col, jnp.int32)], x)
                return carry

            lax.fori_loop(0, G, group, 0)

        def start_writeback(slot):
            _, _, _, out, ofs = sched[slot]
            pltpu.async_copy(rows[slot % NBUF], out.at[pl.ds(base + ofs, CH)],
                             wsems[slot % NBUF])

        def drain_writeback(slot):
            _, _, _, out, ofs = sched[slot]
            pltpu.make_async_copy(rows[slot % NBUF], out.at[pl.ds(base + ofs, CH)],
                                  wsems[slot % NBUF]).wait()

        for s in range(min(NBUF - 1, total)):
            fire(s)
        for s in range(total):
            drain_gather(s)
            if s >= NBUF:
                drain_writeback(s - NBUF)
            extract(s)
            start_writeback(s)
            nxt = s + NBUF - 1
            if nxt < total:
                fire(nxt)
        for s in range(max(total - NBUF, 0), total):
            drain_writeback(s)

    return sc_gather3, NW, b_per_w


def kernel(user, pos, neg, user_table, item_table):
    info = plsc.get_sparse_core_info()
    fn, nw, bw = _build(info.num_cores, info.num_subcores)

    def split(idx):
        idx = idx.astype(jnp.int32)
        return ((idx >> 1).reshape(nw, bw),
                ((idx & 1) * DIM).reshape(nw, bw))

    u_p, u_h = split(user)
    p_p, p_h = split(pos)
    n_p, n_h = split(neg)
    ut2 = user_table.reshape(user_table.shape[0] // 2, 2 * DIM)
    it2 = item_table.reshape(item_table.shape[0] // 2, 2 * DIM)
    return fn(u_p, u_h, p_p, p_h, n_p, n_h, ut2, it2)


# speedup vs baseline: 1.6091x; 1.6091x over previous
import functools

import jax
import jax.numpy as jnp
from jax import lax
from jax.experimental import pallas as pl
from jax.experimental.pallas import tpu as pltpu, tpu_sc as plsc

B = 16384
DIM = 64
CH = 128
HF = CH // 2
NBUF = 3


@functools.lru_cache(maxsize=None)
def _build(num_cores, num_subcores):
    NW = num_cores * num_subcores
    b_per_w = B // NW
    NCH = b_per_w // CH
    G = CH // 16

    mesh = plsc.VectorSubcoreMesh(core_axis_name="c", subcore_axis_name="s")
    out_sds = jax.ShapeDtypeStruct((B, DIM), jnp.float32)

    @functools.partial(
        pl.kernel,
        mesh=mesh,
        out_type=(out_sds, out_sds, out_sds),
        scratch_types=[
            pltpu.VMEM((b_per_w,), jnp.int32),
            pltpu.VMEM((b_per_w,), jnp.int32),
            pltpu.VMEM((b_per_w,), jnp.int32),
            [pltpu.VMEM((HF, DIM), jnp.float32) for _ in range(NBUF)],
            [pltpu.VMEM_SHARED((NW, HF, DIM), jnp.float32) for _ in range(NBUF)],
            [pltpu.SemaphoreType.DMA for _ in range(NBUF)],
            [pltpu.SemaphoreType.DMA for _ in range(NBUF)],
            [pltpu.SemaphoreType.DMA for _ in range(NBUF)],
            [pltpu.SemaphoreType.DMA for _ in range(NBUF)],
        ],
    )
    def sc_gather3(u_i, p_i, n_i, utab, itab, out_u, out_p, out_n,
                   uidx, pidx, nidx, vbufs, sbufs, vgs, sgs, vws, sws):
        wid = lax.axis_index("s") * num_cores + lax.axis_index("c")
        base = wid * b_per_w

        pltpu.sync_copy(u_i.at[wid], uidx)
        pltpu.sync_copy(p_i.at[wid], pidx)
        pltpu.sync_copy(n_i.at[wid], nidx)

        sched = []
        for tab, idx, out in ((utab, uidx, out_u),
                              (itab, pidx, out_p),
                              (itab, nidx, out_n)):
            for c in range(NCH):
                sched.append((tab, idx, out, c * CH))
        total = len(sched)

        def fire(slot):
            tab, idx, _, ofs = sched[slot]
            vbuf = vbufs[slot % NBUF]
            sbuf = sbufs[slot % NBUF].at[wid]

            def issue(g, carry):
                v_lo = idx[pl.ds(ofs + g * 16, 16)]
                v_hi = idx[pl.ds(ofs + HF + g * 16, 16)]
                for l in range(16):
                    j = g * 16 + l
                    pltpu.async_copy(tab.at[pl.ds(v_lo[l], 1)],
                                     vbuf.at[pl.ds(j, 1)],
                                     vgs[slot % NBUF])
                    pltpu.async_copy(tab.at[pl.ds(v_hi[l], 1)],
                                     sbuf.at[pl.ds(j, 1)],
                                     sgs[slot % NBUF])
                return carry

            lax.fori_loop(0, G // 2, issue, 0)

        def drain_gathers(slot):
            tab = sched[slot][0]
            vbuf = vbufs[slot % NBUF]
            sbuf = sbufs[slot % NBUF].at[wid]

            def one(j, carry):
                pltpu.make_async_copy(tab.at[pl.ds(0, 1)],
                                      vbuf.at[pl.ds(0, 1)],
                                      vgs[slot % NBUF]).wait()
                pltpu.make_async_copy(tab.at[pl.ds(0, 1)],
                                      sbuf.at[pl.ds(0, 1)],
                                      sgs[slot % NBUF]).wait()
                return carry

            lax.fori_loop(0, HF, one, 0)

        def start_writeback(slot):
            _, _, out, ofs = sched[slot]
            pltpu.async_copy(vbufs[slot % NBUF],
                             out.at[pl.ds(base + ofs, HF)], vws[slot % NBUF])
            pltpu.async_copy(sbufs[slot % NBUF].at[wid],
                             out.at[pl.ds(base + ofs + HF, HF)], sws[slot % NBUF])

        def drain_writeback(slot):
            _, _, out, ofs = sched[slot]
            pltpu.make_async_copy(vbufs[slot % NBUF],
                                  out.at[pl.ds(base + ofs, HF)],
                                  vws[slot % NBUF]).wait()
            pltpu.make_async_copy(sbufs[slot % NBUF].at[wid],
                                  out.at[pl.ds(base + ofs + HF, HF)],
                                  sws[slot % NBUF]).wait()

        for s in range(min(NBUF - 1, total)):
            fire(s)
        for s in range(total):
            drain_gathers(s)
            if s >= NBUF:
                drain_writeback(s - NBUF)
            start_writeback(s)
            nxt = s + NBUF - 1
            if nxt < total:
                fire(nxt)
        for s in range(max(total - NBUF, 0), total):
            drain_writeback(s)

    return sc_gather3, NW, b_per_w


def kernel(user, pos, neg, user_table, item_table):
    info = plsc.get_sparse_core_info()
    fn, nw, bw = _build(info.num_cores, info.num_subcores)
    u = user.astype(jnp.int32).reshape(nw, bw)
    p = pos.astype(jnp.int32).reshape(nw, bw)
    n = neg.astype(jnp.int32).reshape(nw, bw)
    return fn(u, p, n, user_table, item_table)
